# baseline (speedup 1.0000x reference)
import functools

import jax
import jax.numpy as jnp
from jax import lax
from jax.experimental import pallas as pl
from jax.experimental.pallas import tpu as pltpu
from jax.experimental.pallas import tpu_sc as plsc

NW = 32
CHUNK = 4


def _sc_gather(table, table_flat, idx3, pidx2):
    n_chunks = idx3.shape[1]
    v = table.shape[1]
    b_per_w = n_chunks * CHUNK
    n = NW * b_per_w
    mesh = plsc.VectorSubcoreMesh(core_axis_name="c", subcore_axis_name="s")

    @functools.partial(
        pl.kernel,
        mesh=mesh,
        out_type=(
            jax.ShapeDtypeStruct((n, v), jnp.float32),
            jax.ShapeDtypeStruct((n,), jnp.float32),
        ),
        scratch_types=[
            pltpu.VMEM((n_chunks, CHUNK), jnp.int32),
            pltpu.VMEM((b_per_w,), jnp.int32),
            pltpu.VMEM((b_per_w,), jnp.float32),
            pltpu.VMEM((2, CHUNK, v), jnp.float32),
            pltpu.SemaphoreType.DMA,
            pltpu.SemaphoreType.DMA,
        ],
    )
    def k(table_hbm, tflat_hbm, idx_hbm, pidx_hbm, out_hbm, picked_hbm,
          idx_v, pidx_v, picked_v, rows_v, gsem, wsem):
        cid = lax.axis_index("c")
        sid = lax.axis_index("s")
        wid = sid * 2 + cid
        base = wid * b_per_w
        pltpu.sync_copy(idx_hbm.at[wid], idx_v)
        pltpu.sync_copy(pidx_hbm.at[wid], pidx_v)
        pk = pltpu.async_copy(tflat_hbm.at[pidx_v], picked_v, wsem)

        def gather(c):
            return pltpu.async_copy(
                table_hbm.at[idx_v.at[c]], rows_v.at[c % 2], gsem
            )

        def write(c):
            return pltpu.async_copy(
                rows_v.at[c % 2], out_hbm.at[pl.ds(base + c * CHUNK, CHUNK)], wsem
            )

        g = [None] * n_chunks
        w = [None] * n_chunks
        g[0] = gather(0)
        pk.wait()
        pltpu.sync_copy(picked_v, picked_hbm.at[pl.ds(base, b_per_w)])
        for c in range(n_chunks):
            g[c].wait()
            w[c] = write(c)
            if c + 1 < n_chunks:
                if c >= 1:
                    w[c - 1].wait()
                g[c + 1] = gather(c + 1)
        w[n_chunks - 2].wait()
        w[n_chunks - 1].wait()

    return k(table, table_flat, idx3, pidx2)


def _tc_loss(logits2, picked):
    n, v = logits2.shape
    rows = 128
    g = n // rows
    pk3 = picked.reshape(g, 1, rows)

    def body(lg_ref, pk_ref, loss_ref):
        i = pl.program_id(0)
        x = lg_ref[...]
        m = jnp.max(x, axis=1, keepdims=True)
        s = jnp.sum(jnp.exp(x - m), axis=1)
        lse = m[:, 0] + jnp.log(s)
        part = jnp.sum(lse - pk_ref[0, 0, :])

        @pl.when(i == 0)
        def _():
            loss_ref[0, 0] = 0.0

        loss_ref[0, 0] += part

        @pl.when(i == g - 1)
        def _():
            loss_ref[0, 0] = loss_ref[0, 0] * (1.0 / n)

    loss = pl.pallas_call(
        body,
        grid=(g,),
        in_specs=[
            pl.BlockSpec((rows, v), lambda i: (i, 0)),
            pl.BlockSpec((1, 1, rows), lambda i: (i, 0, 0)),
        ],
        out_specs=pl.BlockSpec(memory_space=pltpu.SMEM),
        out_shape=jax.ShapeDtypeStruct((1, 1), jnp.float32),
    )(logits2, pk3)
    return loss[0, 0]


def kernel(table, idx, targets):
    n = idx.size
    v = table.shape[1]
    b_per_w = n // NW
    idx32 = idx.reshape(-1).astype(jnp.int32)
    tg32 = targets.reshape(-1).astype(jnp.int32)
    idx3 = idx32.reshape(NW, b_per_w // CHUNK, CHUNK)
    pidx2 = (idx32 * v + tg32).reshape(NW, b_per_w)
    logits2, picked = _sc_gather(table, table.reshape(-1), idx3, pidx2)
    loss = _tc_loss(logits2, picked)
    return (logits2, loss)


# speedup vs baseline: 2.0769x; 2.0769x over previous
import functools

import jax
import jax.numpy as jnp
from jax import lax
from jax.experimental import pallas as pl
from jax.experimental.pallas import tpu as pltpu
from jax.experimental.pallas import tpu_sc as plsc

NW = 32
CHUNK = 4


def _sc_gather(table, idx3):
    n_chunks = idx3.shape[1]
    v = table.shape[1]
    b_per_w = n_chunks * CHUNK
    n = NW * b_per_w
    mesh = plsc.VectorSubcoreMesh(core_axis_name="c", subcore_axis_name="s")

    nbuf = 3

    @functools.partial(
        pl.kernel,
        mesh=mesh,
        out_type=jax.ShapeDtypeStruct((n, v), jnp.float32),
        scratch_types=[
            pltpu.VMEM((n_chunks, CHUNK), jnp.int32),
            pltpu.VMEM((nbuf, CHUNK, v), jnp.float32),
            pltpu.SemaphoreType.DMA,
            pltpu.SemaphoreType.DMA,
        ],
    )
    def k(table_hbm, idx_hbm, out_hbm, idx_v, rows_v, gsem, wsem):
        cid = lax.axis_index("c")
        sid = lax.axis_index("s")
        wid = sid * 2 + cid
        base = wid * b_per_w
        pltpu.sync_copy(idx_hbm.at[wid], idx_v)

        def gather(c):
            return pltpu.async_copy(
                table_hbm.at[idx_v.at[c]], rows_v.at[c % nbuf], gsem
            )

        def write(c):
            return pltpu.async_copy(
                rows_v.at[c % nbuf], out_hbm.at[pl.ds(base + c * CHUNK, CHUNK)], wsem
            )

        g = [None] * n_chunks
        w = [None] * n_chunks
        g[0] = gather(0)
        g[1] = gather(1)
        for c in range(n_chunks):
            g[c].wait()
            w[c] = write(c)
            if c + 2 < n_chunks:
                if c >= 1:
                    w[c - 1].wait()
                g[c + 2] = gather(c + 2)
        w[n_chunks - 3].wait()
        w[n_chunks - 2].wait()
        w[n_chunks - 1].wait()

    return k(table, idx3)


def _tc_loss(logits2, flat_tg):
    n, v = logits2.shape
    rows = 128
    g = n // rows
    tg3 = flat_tg.reshape(g, 1, rows)

    def body(lg_ref, tg_ref, loss_ref):
        i = pl.program_id(0)
        x = lg_ref[...]
        m = jnp.max(x, axis=1, keepdims=True)
        s = jnp.sum(jnp.exp(x - m), axis=1)
        lse = m[:, 0] + jnp.log(s)
        tg = tg_ref[0, 0, :]
        col = lax.broadcasted_iota(jnp.int32, (rows, v), 1)
        picked = jnp.sum(jnp.where(col == tg[:, None], x, 0.0), axis=1)
        part = jnp.sum(lse - picked)

        @pl.when(i == 0)
        def _():
            loss_ref[0, 0] = 0.0

        loss_ref[0, 0] += part

        @pl.when(i == g - 1)
        def _():
            loss_ref[0, 0] = loss_ref[0, 0] * (1.0 / n)

    loss = pl.pallas_call(
        body,
        grid=(g,),
        in_specs=[
            pl.BlockSpec((rows, v), lambda i: (i, 0)),
            pl.BlockSpec((1, 1, rows), lambda i: (i, 0, 0)),
        ],
        out_specs=pl.BlockSpec(memory_space=pltpu.SMEM),
        out_shape=jax.ShapeDtypeStruct((1, 1), jnp.float32),
    )(logits2, tg3)
    return loss[0, 0]


def kernel(table, idx, targets):
    n = idx.size
    v = table.shape[1]
    b_per_w = n // NW
    del v
    idx32 = idx.reshape(-1).astype(jnp.int32)
    tg32 = targets.reshape(-1).astype(jnp.int32)
    idx3 = idx32.reshape(NW, b_per_w // CHUNK, CHUNK)
    logits2 = _sc_gather(table, idx3)
    loss = _tc_loss(logits2, tg32)
    return (logits2, loss)
